# baseline (speedup 1.0000x reference)
import functools

import jax
import jax.numpy as jnp
from jax import lax
from jax.experimental import pallas as pl
from jax.experimental.pallas import tpu as pltpu
from jax.experimental.pallas import tpu_sc as plsc


def _gather_rows(table, idx):
    batch, = idx.shape
    _, dim = table.shape
    info = plsc.get_sparse_core_info()
    nw = info.num_cores * info.num_subcores
    b_per_w = batch // nw
    mesh = plsc.VectorSubcoreMesh(core_axis_name="c", subcore_axis_name="s")

    @functools.partial(
        pl.kernel,
        mesh=mesh,
        out_type=jax.ShapeDtypeStruct((batch, dim), jnp.float32),
        scratch_types=[
            pltpu.VMEM((b_per_w,), jnp.int32),
            pltpu.VMEM((b_per_w, dim), jnp.float32),
            pltpu.SemaphoreType.DMA,
        ],
        compiler_params=pltpu.CompilerParams(use_tc_tiling_on_sc=False),
    )
    def gather_kernel(table_hbm, idx_hbm, out_hbm, idx_v, rows_v, sem):
        wid = lax.axis_index("s") * info.num_cores + lax.axis_index("c")
        base = wid * b_per_w
        pltpu.sync_copy(idx_hbm.at[pl.ds(base, b_per_w)], idx_v)
        pltpu.async_copy(table_hbm.at[idx_v], rows_v, sem).wait()
        pltpu.sync_copy(rows_v, out_hbm.at[pl.ds(base, b_per_w)])

    return gather_kernel(table, idx)


def _mlp_body(h_ref, w1_ref, b1_ref, w2_ref, b2_ref, o_ref):
    h = h_ref[...]
    z = lax.dot_general(h, w1_ref[...], (((1,), (1,)), ((), ())),
                        preferred_element_type=jnp.float32)
    z = jnp.maximum(z + b1_ref[...], 0.0)
    o_ref[...] = lax.dot_general(z, w2_ref[...], (((1,), (1,)), ((), ())),
                                 preferred_element_type=jnp.float32) + b2_ref[...]


def _mlp(h, W1, b1, W2, b2):
    batch = h.shape[0]
    out_dim = W2.shape[0]
    return pl.pallas_call(
        _mlp_body,
        out_shape=jax.ShapeDtypeStruct((batch, out_dim), jnp.float32),
    )(h, W1, b1.reshape(1, -1), W2, b2.reshape(1, -1))


def kernel(x, emb, W1, b1, W2, b2):
    h = _gather_rows(emb, x.astype(jnp.int32))
    return _mlp(h, W1, b1, W2, b2)


# speedup vs baseline: 1.3148x; 1.3148x over previous
import functools

import jax
import jax.numpy as jnp
from jax import lax
from jax.experimental import pallas as pl
from jax.experimental.pallas import tpu as pltpu

_CHUNK = 128


def _body(idx_ref, emb_ref, w1_ref, b1_ref, w2_ref, b2_ref, o_ref,
          h0, h1, sem0, sem1):
    i = pl.program_id(0)
    n = pl.num_programs(0)

    def issue(chunk, buf, sem):
        for j in range(_CHUNK):
            r = idx_ref[chunk * _CHUNK + j]
            pltpu.make_async_copy(emb_ref.at[r], buf.at[j], sem).start()

    def drain(buf, sem):
        for j in range(_CHUNK):
            pltpu.make_async_copy(emb_ref.at[0], buf.at[j], sem).wait()

    @pl.when(i == 0)
    def _prologue():
        issue(0, h0, sem0)

    @pl.when(jnp.logical_and(i < n - 1, (i + 1) % 2 == 0))
    def _issue_next_even():
        issue(i + 1, h0, sem0)

    @pl.when(jnp.logical_and(i < n - 1, (i + 1) % 2 == 1))
    def _issue_next_odd():
        issue(i + 1, h1, sem1)

    @pl.when(i % 2 == 0)
    def _drain_even():
        drain(h0, sem0)

    @pl.when(i % 2 == 1)
    def _drain_odd():
        drain(h1, sem1)

    h = jnp.where((i % 2 == 0), h0[...], h1[...])

    z = lax.dot_general(h, w1_ref[...], (((1,), (1,)), ((), ())),
                        preferred_element_type=jnp.float32)
    z = jnp.maximum(z + b1_ref[...], 0.0)
    o_ref[...] = lax.dot_general(z, w2_ref[...], (((1,), (1,)), ((), ())),
                                 preferred_element_type=jnp.float32) + b2_ref[...]


def kernel(x, emb, W1, b1, W2, b2):
    batch, = x.shape
    _, dim = emb.shape
    hidden = W1.shape[0]
    out_dim = W2.shape[0]
    n_chunks = batch // _CHUNK

    grid_spec = pltpu.PrefetchScalarGridSpec(
        num_scalar_prefetch=1,
        grid=(n_chunks,),
        in_specs=[
            pl.BlockSpec(memory_space=pltpu.HBM),
            pl.BlockSpec((hidden, dim), lambda i, s: (0, 0)),
            pl.BlockSpec((1, hidden), lambda i, s: (0, 0)),
            pl.BlockSpec((out_dim, hidden), lambda i, s: (0, 0)),
            pl.BlockSpec((1, out_dim), lambda i, s: (0, 0)),
        ],
        out_specs=pl.BlockSpec((_CHUNK, out_dim), lambda i, s: (i, 0)),
        scratch_shapes=[
            pltpu.VMEM((_CHUNK, dim), jnp.float32),
            pltpu.VMEM((_CHUNK, dim), jnp.float32),
            pltpu.SemaphoreType.DMA,
            pltpu.SemaphoreType.DMA,
        ],
    )
    return pl.pallas_call(
        _body,
        grid_spec=grid_spec,
        out_shape=jax.ShapeDtypeStruct((batch, out_dim), jnp.float32),
    )(x.astype(jnp.int32), emb, W1, b1.reshape(1, -1), W2, b2.reshape(1, -1))
